# baseline (speedup 1.0000x reference)
import functools

import jax
import jax.numpy as jnp
from jax import lax
from jax.experimental import pallas as pl
from jax.experimental.pallas import tpu as pltpu
from jax.experimental.pallas import tpu_sc as plsc

HID = 64
PADW = 128
M1, M2 = 47, 11
NC, NS, LANES = 2, 16, 16
NW = NC * NS
CHUNK = 128
NBUF = 2


def _embed(ids3, pos3, sub_p, m1_p, m2_p, n_b, n_l):
    n_chunks = n_l
    n_rounds = n_chunks // NBUF
    mesh = plsc.VectorSubcoreMesh(
        core_axis_name="c", subcore_axis_name="s", num_cores=NC, num_subcores=NS
    )

    @functools.partial(
        pl.kernel,
        out_type=jax.ShapeDtypeStruct((n_l, HID, n_b), jnp.float32),
        mesh=mesh,
        compiler_params=pltpu.CompilerParams(use_tc_tiling_on_sc=True,
                                             needs_layout_passes=False),
        scratch_types=[
            pltpu.VMEM((n_chunks, CHUNK), jnp.int32),
            pltpu.VMEM((n_chunks, CHUNK), jnp.int32),
            pltpu.VMEM((NBUF, CHUNK, PADW), jnp.float32),
            pltpu.VMEM((NBUF, HID, CHUNK), jnp.float32),
            pltpu.VMEM((M1, PADW), jnp.float32),
            pltpu.VMEM((M2, PADW), jnp.float32),
            pltpu.VMEM((HID, 64), jnp.float32),
            pltpu.SemaphoreType.DMA,
        ] + [pltpu.SemaphoreType.DMA] * (2 * NBUF),
    )
    def run(ids_hbm, pos_hbm, sub_hbm, m1_hbm, m2_hbm, out_hbm,
            ids_v, pos_v, rows, trans, m1_v, m2_v, fused_t,
            sem_i, *sems):
        sem_g = sems[0:NBUF]
        sem_o = sems[NBUF:2 * NBUF]
        cid = lax.axis_index("c")
        sid = lax.axis_index("s")
        wid = sid * NC + cid

        cp_ids = pltpu.async_copy(ids_hbm.at[wid], ids_v, sem_i)
        cp_pos = pltpu.async_copy(pos_hbm.at[wid], pos_v, sem_i)

        pltpu.sync_copy(m1_hbm, m1_v)
        pltpu.sync_copy(m2_hbm, m2_v)
        lanes_iota = lax.iota(jnp.int32, LANES)
        for p in range(50):
            pcol = jnp.full((LANES,), p, jnp.int32)
            for cg in range(HID // LANES):
                sl = pl.ds(cg * LANES, LANES)
                v = m1_v[p % M1, sl] + m2_v[p % M2, sl]
                plsc.store_scatter(fused_t, [lanes_iota + cg * LANES, pcol], v)

        cp_ids.wait()
        cp_pos.wait()

        def start_g(i, b):
            return pltpu.async_copy(sub_hbm.at[ids_v.at[i]], rows.at[b],
                                    sem_g[b])

        def wait_g(i, b):
            pltpu.make_async_copy(sub_hbm.at[ids_v.at[i]], rows.at[b],
                                  sem_g[b]).wait()

        def start_o(i, b):
            return pltpu.async_copy(
                trans.at[b], out_hbm.at[i, :, pl.ds(wid * CHUNK, CHUNK)],
                sem_o[b])

        def wait_o(i, b):
            pltpu.make_async_copy(
                trans.at[b], out_hbm.at[i, :, pl.ds(wid * CHUNK, CHUNK)],
                sem_o[b]).wait()

        row_idx = [lanes_iota + bg * LANES for bg in range(CHUNK // LANES)]

        def transpose_add(i, b):
            rows_b = rows.at[b]
            trans_b = trans.at[b]
            pvs = [pos_v[i, pl.ds(bg * LANES, LANES)]
                   for bg in range(CHUNK // LANES)]

            def col_body(c, carry):
                ccol = jnp.full((LANES,), c, jnp.int32)
                for bg in range(CHUNK // LANES):
                    v = plsc.load_gather(rows_b, [row_idx[bg], ccol])
                    pv = plsc.load_gather(fused_t, [ccol, pvs[bg]])
                    trans_b[c, pl.ds(bg * LANES, LANES)] = v + pv
                return carry

            lax.fori_loop(0, HID, col_body, 0)

        def do_round(i0, prime_next):
            for b in range(NBUF):
                wait_g(i0 + b, b)
                transpose_add(i0 + b, b)
                start_o(i0 + b, b)
            for b in range(NBUF):
                wait_o(i0 + b, b)
                if prime_next:
                    start_g(i0 + NBUF + b, b)

        for b in range(NBUF):
            start_g(b, b)

        def round_body(r, carry):
            do_round(r * NBUF, prime_next=True)
            return carry

        lax.fori_loop(0, n_rounds - 1, round_body, 0)
        do_round((n_rounds - 1) * NBUF, prime_next=False)

    return run(ids3, pos3, sub_p, m1_p, m2_p)


def kernel(input_ids, position_ids, pretrain_embed, subword_table, m1_table, m2_table):
    b, l = input_ids.shape
    ids3 = input_ids.T.reshape(l, NW, CHUNK).transpose(1, 0, 2).astype(jnp.int32)
    pos3 = position_ids.T.reshape(l, NW, CHUNK).transpose(1, 0, 2).astype(jnp.int32)
    padc = ((0, 0), (0, PADW - HID))
    sub_p = jnp.pad(subword_table, padc)
    m1_p = jnp.pad(m1_table, padc)
    m2_p = jnp.pad(m2_table, padc)
    out = _embed(ids3, pos3, sub_p, m1_p, m2_p, b, l)
    return out.transpose(2, 0, 1)


# speedup vs baseline: 1.1958x; 1.1958x over previous
import functools

import jax
import jax.numpy as jnp
from jax import lax
from jax.experimental import pallas as pl
from jax.experimental.pallas import tpu as pltpu
from jax.experimental.pallas import tpu_sc as plsc

HID = 64
PADW = 128
M1, M2 = 47, 11
LMAX = 50
NC, NS, LANES = 2, 16, 16
NW = NC * NS
CHUNK = 128
NBUF = 5


def _embed(ids, pos, sub_p, m1_p, m2_p):
    n = ids.size
    n_per_w = n // NW
    n_chunks = n_per_w // CHUNK
    n_rounds = n_chunks // NBUF
    mesh = plsc.VectorSubcoreMesh(
        core_axis_name="c", subcore_axis_name="s", num_cores=NC, num_subcores=NS
    )

    @functools.partial(
        pl.kernel,
        out_type=jax.ShapeDtypeStruct((n, PADW), jnp.float32),
        mesh=mesh,
        compiler_params=pltpu.CompilerParams(use_tc_tiling_on_sc=True),
        scratch_types=[
            pltpu.VMEM((n_chunks, CHUNK), jnp.int32),
            pltpu.VMEM((n_chunks, CHUNK), jnp.int32),
            pltpu.VMEM((NBUF, CHUNK, PADW), jnp.float32),
            pltpu.VMEM((M1, PADW), jnp.float32),
            pltpu.VMEM((M2, PADW), jnp.float32),
            pltpu.VMEM((LMAX, PADW), jnp.float32),
            pltpu.VMEM_SHARED((LMAX, PADW), jnp.float32),
            pltpu.SemaphoreType.DMA,
        ] + [pltpu.SemaphoreType.DMA] * (3 * NBUF),
    )
    def run(ids_hbm, pos_hbm, sub_hbm, m1_hbm, m2_hbm, out_hbm,
            ids_v, pos_v, rows, m1_v, m2_v, fused_v, fused_sh,
            sem_i, *sems):
        sem_g = sems[0:NBUF]
        sem_p = sems[NBUF:2 * NBUF]
        sem_o = sems[2 * NBUF:3 * NBUF]
        cid = lax.axis_index("c")
        sid = lax.axis_index("s")
        wid = sid * NC + cid
        base = wid * n_per_w

        cp_ids = pltpu.async_copy(ids_hbm.at[wid], ids_v, sem_i)
        cp_pos = pltpu.async_copy(pos_hbm.at[wid], pos_v, sem_i)

        @pl.when(sid == 0)
        def _build():
            pltpu.sync_copy(m1_hbm, m1_v)
            pltpu.sync_copy(m2_hbm, m2_v)
            for p in range(LMAX):
                for j in range(PADW // LANES):
                    sl = pl.ds(j * LANES, LANES)
                    fused_v[p, sl] = m1_v[p % M1, sl] + m2_v[p % M2, sl]
            pltpu.sync_copy(fused_v, fused_sh)

        plsc.subcore_barrier()
        cp_ids.wait()
        cp_pos.wait()

        def start_g(i, b):
            return pltpu.async_copy(sub_hbm.at[ids_v.at[i]], rows.at[b],
                                    sem_g[b])

        def wait_g(i, b):
            pltpu.make_async_copy(sub_hbm.at[ids_v.at[i]], rows.at[b],
                                  sem_g[b]).wait()

        def start_p(i, b):
            return pltpu.async_copy(fused_sh.at[pos_v.at[i]], rows.at[b],
                                    sem_p[b], add=True)

        def start_o(i, b):
            return pltpu.async_copy(rows.at[b],
                                    out_hbm.at[pl.ds(base + i * CHUNK, CHUNK)],
                                    sem_o[b])

        def do_round(i0, prime_next):
            pd = []
            for b in range(NBUF):
                wait_g(i0 + b, b)
                pd.append(start_p(i0 + b, b))
            od = []
            for b in range(NBUF):
                pd[b].wait()
                od.append(start_o(i0 + b, b))
            for b in range(NBUF):
                od[b].wait()
                if prime_next:
                    start_g(i0 + NBUF + b, b)

        for b in range(NBUF):
            start_g(b, b)

        def round_body(r, carry):
            do_round(r * NBUF, prime_next=True)
            return carry

        lax.fori_loop(0, n_rounds - 1, round_body, 0)
        do_round((n_rounds - 1) * NBUF, prime_next=False)

    return run(ids, pos, sub_p, m1_p, m2_p)


def kernel(input_ids, position_ids, pretrain_embed, subword_table, m1_table, m2_table):
    b, l = input_ids.shape
    n = b * l
    n_per_w = n // NW
    n_chunks = n_per_w // CHUNK
    ids = input_ids.reshape(NW, n_chunks, CHUNK).astype(jnp.int32)
    pos = position_ids.reshape(NW, n_chunks, CHUNK).astype(jnp.int32)
    padc = ((0, 0), (0, PADW - HID))
    sub_p = jnp.pad(subword_table, padc)
    m1_p = jnp.pad(m1_table, padc)
    m2_p = jnp.pad(m2_table, padc)
    out = _embed(ids, pos, sub_p, m1_p, m2_p)
    return out[:, :HID].reshape(b, l, HID)


# speedup vs baseline: 1.2213x; 1.0213x over previous
import functools

import jax
import jax.numpy as jnp
from jax import lax
from jax.experimental import pallas as pl
from jax.experimental.pallas import tpu as pltpu
from jax.experimental.pallas import tpu_sc as plsc

HID = 64
PADW = 128
M1, M2 = 47, 11
LMAX = 50
NC, NS, LANES = 2, 16, 16
NW = NC * NS
CHUNK = 128
NBUF = 5


def _embed(ids, pos, sub_p, m1_p, m2_p):
    n = ids.size
    n_per_w = n // NW
    n_chunks = n_per_w // CHUNK
    n_rounds = n_chunks // NBUF
    mesh = plsc.VectorSubcoreMesh(
        core_axis_name="c", subcore_axis_name="s", num_cores=NC, num_subcores=NS
    )

    @functools.partial(
        pl.kernel,
        out_type=jax.ShapeDtypeStruct((n, HID), jnp.float32),
        mesh=mesh,
        compiler_params=pltpu.CompilerParams(use_tc_tiling_on_sc=False),
        scratch_types=[
            pltpu.VMEM((n_chunks, CHUNK), jnp.int32),
            pltpu.VMEM((n_chunks, CHUNK), jnp.int32),
            pltpu.VMEM((NBUF, CHUNK, PADW), jnp.float32),
            pltpu.VMEM((M1, PADW), jnp.float32),
            pltpu.VMEM((M2, PADW), jnp.float32),
            pltpu.VMEM((LMAX, PADW), jnp.float32),
            pltpu.VMEM_SHARED((LMAX, PADW), jnp.float32),
            pltpu.SemaphoreType.DMA,
        ] + [pltpu.SemaphoreType.DMA] * (3 * NBUF),
    )
    def run(ids_hbm, pos_hbm, sub_hbm, m1_hbm, m2_hbm, out_hbm,
            ids_v, pos_v, rows, m1_v, m2_v, fused_v, fused_sh,
            sem_i, *sems):
        sem_g = sems[0:NBUF]
        sem_p = sems[NBUF:2 * NBUF]
        sem_o = sems[2 * NBUF:3 * NBUF]
        cid = lax.axis_index("c")
        sid = lax.axis_index("s")
        wid = sid * NC + cid
        base = wid * n_per_w

        cp_ids = pltpu.async_copy(ids_hbm.at[wid], ids_v, sem_i)
        cp_pos = pltpu.async_copy(pos_hbm.at[wid], pos_v, sem_i)

        @pl.when(sid == 0)
        def _build():
            pltpu.sync_copy(m1_hbm, m1_v)
            pltpu.sync_copy(m2_hbm, m2_v)
            for p in range(LMAX):
                for j in range(PADW // LANES):
                    sl = pl.ds(j * LANES, LANES)
                    fused_v[p, sl] = m1_v[p % M1, sl] + m2_v[p % M2, sl]
            pltpu.sync_copy(fused_v, fused_sh)

        plsc.subcore_barrier()
        cp_ids.wait()
        cp_pos.wait()

        def start_g(i, b):
            return pltpu.async_copy(sub_hbm.at[ids_v.at[i]], rows.at[b],
                                    sem_g[b])

        def wait_g(i, b):
            pltpu.make_async_copy(sub_hbm.at[ids_v.at[i]], rows.at[b],
                                  sem_g[b]).wait()

        def start_p(i, b):
            return pltpu.async_copy(fused_sh.at[pos_v.at[i]], rows.at[b],
                                    sem_p[b], add=True)

        def start_o(i, b):
            return pltpu.async_copy(rows.at[b, :, pl.ds(0, HID)],
                                    out_hbm.at[pl.ds(base + i * CHUNK, CHUNK)],
                                    sem_o[b])

        def do_round(i0, prime_next):
            pd = []
            for b in range(NBUF):
                wait_g(i0 + b, b)
                pd.append(start_p(i0 + b, b))
            od = []
            for b in range(NBUF):
                pd[b].wait()
                od.append(start_o(i0 + b, b))
            for b in range(NBUF):
                od[b].wait()
                if prime_next:
                    start_g(i0 + NBUF + b, b)

        for b in range(NBUF):
            start_g(b, b)

        def round_body(r, carry):
            do_round(r * NBUF, prime_next=True)
            return carry

        lax.fori_loop(0, n_rounds - 1, round_body, 0)
        do_round((n_rounds - 1) * NBUF, prime_next=False)

    return run(ids, pos, sub_p, m1_p, m2_p)


def kernel(input_ids, position_ids, pretrain_embed, subword_table, m1_table, m2_table):
    b, l = input_ids.shape
    n = b * l
    n_per_w = n // NW
    n_chunks = n_per_w // CHUNK
    ids = input_ids.reshape(NW, n_chunks, CHUNK).astype(jnp.int32)
    pos = position_ids.reshape(NW, n_chunks, CHUNK).astype(jnp.int32)
    padc = ((0, 0), (0, PADW - HID))
    sub_p = jnp.pad(subword_table, padc)
    m1_p = jnp.pad(m1_table, padc)
    m2_p = jnp.pad(m2_table, padc)
    out = _embed(ids, pos, sub_p, m1_p, m2_p)
    return out.reshape(b, l, HID)
